# baseline (speedup 1.0000x reference)
import functools

import jax
import jax.numpy as jnp
from jax.experimental import pallas as pl
from jax.experimental.pallas import tpu as pltpu


T = 8192
D_IN = 2048
D_OUT = 2048
MAX_LORAS = 8
RANK = 16
LR = MAX_LORAS * RANK

BM = 512


def _fused_body(idx_ref, x_ref, w_ref, a_ref, b3_ref, bias_ref, out_ref,
                b_ref):
    @pl.when(pl.program_id(0) == 0)
    def _transpose_b():
        b_ref[...] = jnp.transpose(b3_ref[...], (0, 2, 1)).reshape(LR, D_OUT)

    x = x_ref[...]
    base = jnp.dot(x, w_ref[...], preferred_element_type=jnp.float32)
    shrink = jax.lax.dot_general(
        x, a_ref[...], (((1,), (1,)), ((), ())),
        preferred_element_type=jnp.float32)
    idx = idx_ref[...]
    grp = jax.lax.broadcasted_iota(jnp.int32, (BM, LR), 1) // RANK
    mshrink = jnp.where(grp == idx, shrink, 0.0)
    expand = jnp.dot(mshrink, b_ref[...], preferred_element_type=jnp.float32)
    lane8 = jax.lax.broadcasted_iota(jnp.int32, (BM, MAX_LORAS), 1)
    onehot = (lane8 == idx).astype(jnp.float32)
    bias = jnp.dot(onehot, bias_ref[...], preferred_element_type=jnp.float32)
    out_ref[...] = base + expand + bias


@jax.jit
def kernel(x, token_lora_indices, W, lora_a, lora_b, lora_bias):
    idx = token_lora_indices.astype(jnp.int32).reshape(T, 1)
    a_cat = lora_a.reshape(LR, D_IN)

    grid = (T // BM,)
    return pl.pallas_call(
        _fused_body,
        grid=grid,
        in_specs=[
            pl.BlockSpec((BM, 1), lambda i: (i, 0)),
            pl.BlockSpec((BM, D_IN), lambda i: (i, 0)),
            pl.BlockSpec((D_IN, D_OUT), lambda i: (0, 0)),
            pl.BlockSpec((LR, D_IN), lambda i: (0, 0)),
            pl.BlockSpec((MAX_LORAS, D_OUT, RANK), lambda i: (0, 0, 0)),
            pl.BlockSpec((MAX_LORAS, D_OUT), lambda i: (0, 0)),
        ],
        out_specs=pl.BlockSpec((BM, D_OUT), lambda i: (i, 0)),
        out_shape=jax.ShapeDtypeStruct((T, D_OUT), jnp.float32),
        scratch_shapes=[pltpu.VMEM((LR, D_OUT), jnp.float32)],
    )(idx, x, W, a_cat, lora_b, lora_bias)


# speedup vs baseline: 1.0965x; 1.0965x over previous
import jax
import jax.numpy as jnp
from jax.experimental import pallas as pl


T = 8192
D_IN = 2048
D_OUT = 2048
MAX_LORAS = 8
RANK = 16
LR = MAX_LORAS * RANK

BM = 512


def _fused_body(idx_ref, x_ref, w_ref, a_ref, b_ref, bias_ref, out_ref):
    x = x_ref[...]
    base = jnp.dot(x, w_ref[...], preferred_element_type=jnp.float32)
    shrink = jax.lax.dot_general(
        x, a_ref[...], (((1,), (1,)), ((), ())),
        preferred_element_type=jnp.float32)
    idx = idx_ref[...]
    grp = jax.lax.broadcasted_iota(jnp.int32, (BM, LR), 1) // RANK
    mshrink = jnp.where(grp == idx, shrink, 0.0)
    expand = jnp.dot(mshrink, b_ref[...], preferred_element_type=jnp.float32)
    lane8 = jax.lax.broadcasted_iota(jnp.int32, (BM, MAX_LORAS), 1)
    onehot = (lane8 == idx).astype(jnp.float32)
    bias = jnp.dot(onehot, bias_ref[...], preferred_element_type=jnp.float32)
    out_ref[...] = base + expand + bias


@jax.jit
def kernel(x, token_lora_indices, W, lora_a, lora_b, lora_bias):
    idx = token_lora_indices.astype(jnp.int32).reshape(T, 1)
    a_cat = lora_a.reshape(LR, D_IN)
    b_cat = jnp.transpose(lora_b, (0, 2, 1)).reshape(LR, D_OUT)

    grid = (T // BM,)
    return pl.pallas_call(
        _fused_body,
        grid=grid,
        in_specs=[
            pl.BlockSpec((BM, 1), lambda i: (i, 0)),
            pl.BlockSpec((BM, D_IN), lambda i: (i, 0)),
            pl.BlockSpec((D_IN, D_OUT), lambda i: (0, 0)),
            pl.BlockSpec((LR, D_IN), lambda i: (0, 0)),
            pl.BlockSpec((LR, D_OUT), lambda i: (0, 0)),
            pl.BlockSpec((MAX_LORAS, D_OUT), lambda i: (0, 0)),
        ],
        out_specs=pl.BlockSpec((BM, D_OUT), lambda i: (i, 0)),
        out_shape=jax.ShapeDtypeStruct((T, D_OUT), jnp.float32),
    )(idx, x, W, a_cat, b_cat, lora_bias)
